# Initial kernel scaffold; baseline (speedup 1.0000x reference)
import jax
import jax.numpy as jnp
from jax.experimental import pallas as pl


def kernel(x, h_embed, k_embed, l_embed):
    raise NotImplementedError("write your pallas kernel here")


# speedup vs baseline: 2.9489x; 2.9489x over previous
import functools

import jax
import jax.numpy as jnp
from jax import lax
from jax.experimental import pallas as pl
from jax.experimental.pallas import tpu as pltpu
from jax.experimental.pallas import tpu_sc as plsc

H = 32
K = 32
L = 32
SUB = 64
ED = 3 * SUB
HKL = H * K * L
LANES = 16
KC = 8
ROWS = KC * L
NCHUNK = K // KC


@functools.lru_cache(maxsize=None)
def _build_sc(batch: int):
    info = plsc.get_sparse_core_info()
    nc, ns = info.num_cores, info.num_subcores
    nw = nc * ns
    assert nw == H, (nc, ns)

    mesh = plsc.VectorSubcoreMesh(core_axis_name="c", subcore_axis_name="s")

    @functools.partial(
        pl.kernel,
        mesh=mesh,
        out_type=jax.ShapeDtypeStruct((batch * HKL, ED), jnp.float32),
        scratch_types=[
            pltpu.VMEM((1, SUB), jnp.float32),
            pltpu.VMEM((K, SUB), jnp.float32),
            pltpu.VMEM((L, SUB), jnp.float32),
            pltpu.VMEM((ROWS, ED), jnp.float32),
        ],
    )
    def sc_kernel(h_hbm, k_hbm, l_hbm, out_hbm, hrow_v, kt_v, lt_v, buf_v):
        wid = lax.axis_index("s") * nc + lax.axis_index("c")
        h = wid

        pltpu.sync_copy(h_hbm.at[pl.ds(h, 1)], hrow_v)
        pltpu.sync_copy(k_hbm, kt_v)
        pltpu.sync_copy(l_hbm, lt_v)

        for c in range(SUB // LANES):
            hv = hrow_v[0, pl.ds(c * LANES, LANES)]
            for r in range(ROWS):
                buf_v[r, pl.ds(c * LANES, LANES)] = hv
        for l in range(L):
            for c in range(SUB // LANES):
                lv = lt_v[l, pl.ds(c * LANES, LANES)]
                for rep in range(KC):
                    buf_v[rep * L + l, pl.ds(2 * SUB + c * LANES, LANES)] = lv

        def chunk_body(ci, carry):
            for j in range(KC):
                kk = ci * KC + j
                for c in range(SUB // LANES):
                    kv = kt_v[kk, pl.ds(c * LANES, LANES)]
                    for r in range(L):
                        buf_v[j * L + r, pl.ds(SUB + c * LANES, LANES)] = kv
            base = h * (K * L) + ci * ROWS
            for b in range(batch):
                pltpu.sync_copy(buf_v, out_hbm.at[pl.ds(b * HKL + base, ROWS)])
            return carry

        lax.fori_loop(0, NCHUNK, chunk_body, 0)

    return sc_kernel


def kernel(x, h_embed, k_embed, l_embed):
    batch = x.shape[0]
    out2d = _build_sc(batch)(h_embed, k_embed, l_embed)
    return out2d.reshape(batch, HKL, ED)
